# baseline (speedup 1.0000x reference)
import functools

import jax
import jax.numpy as jnp
from jax import lax
from jax.experimental import pallas as pl
from jax.experimental.pallas import tpu as pltpu
from jax.experimental.pallas import tpu_sc as plsc

_NUM_CORES = 2
_NUM_SUBCORES = 16
_NUM_WORKERS = _NUM_CORES * _NUM_SUBCORES

_CHUNK = 32
_NBUF = 3


@functools.partial(jax.jit, static_argnames=("n_rows", "hidden"))
def _lookup(positions2d, table, *, n_rows, hidden):
    per_w = n_rows // _NUM_WORKERS
    n_chunks = per_w // _CHUNK
    mesh = plsc.VectorSubcoreMesh(core_axis_name="c", subcore_axis_name="s")

    @functools.partial(
        pl.kernel,
        mesh=mesh,
        out_type=jax.ShapeDtypeStruct((n_rows, hidden), jnp.float32),
        scratch_types=(
            [pltpu.VMEM((n_chunks, _CHUNK), jnp.int32)]
            + [pltpu.VMEM((_CHUNK, hidden), jnp.float32)] * _NBUF
            + [pltpu.SemaphoreType.DMA] * (2 * _NBUF)
        ),
    )
    def emb_kernel(idx_hbm, table_hbm, out_hbm, idx_v, *bufs):
        rows = bufs[:_NBUF]
        gsem = bufs[_NBUF:2 * _NBUF]
        osem = bufs[2 * _NBUF:]
        wid = lax.axis_index("s") * _NUM_CORES + lax.axis_index("c")
        base = wid * per_w

        pltpu.sync_copy(idx_hbm.at[wid], idx_v)

        gcp = [None] * n_chunks
        ocp = [None] * n_chunks

        def writeback(g):
            b = g % _NBUF
            gcp[g].wait()
            ocp[g] = pltpu.async_copy(
                rows[b], out_hbm.at[pl.ds(base + g * _CHUNK, _CHUNK)], osem[b])

        for g in range(n_chunks):
            b = g % _NBUF
            if g >= _NBUF:
                ocp[g - _NBUF].wait()
            gcp[g] = pltpu.async_copy(table_hbm.at[idx_v.at[g]], rows[b], gsem[b])
            if g >= 1:
                writeback(g - 1)

        writeback(n_chunks - 1)
        for g in range(max(0, n_chunks - _NBUF), n_chunks):
            ocp[g].wait()

    return emb_kernel(positions2d, table)


def kernel(positions, pos_embedding):
    b, t = positions.shape
    n_rows = b * t
    hidden = pos_embedding.shape[1]
    flat = positions.reshape(n_rows).astype(jnp.int32)
    n_sc = (n_rows * 7 // 10) // 1024 * 1024
    pos2d = flat[:n_sc].reshape(_NUM_WORKERS, n_sc // _NUM_WORKERS // _CHUNK, _CHUNK)
    out_sc = _lookup(pos2d, pos_embedding, n_rows=n_sc, hidden=hidden)
    out_tc = jnp.take(pos_embedding, flat[n_sc:], axis=0)
    out = jnp.concatenate([out_sc, out_tc], axis=0)
    return out.reshape(b, t, hidden)


# speedup vs baseline: 1.9564x; 1.9564x over previous
import functools

import jax
import jax.numpy as jnp
from jax import lax
from jax.experimental import pallas as pl
from jax.experimental.pallas import tpu as pltpu
from jax.experimental.pallas import tpu_sc as plsc

_NUM_CORES = 2
_NUM_SUBCORES = 16
_NUM_WORKERS = _NUM_CORES * _NUM_SUBCORES

_CHUNK = 16
_NBUF = 3
_NSLOT = 4


@functools.partial(jax.jit, static_argnames=("n_rows", "hidden"))
def _lookup(positions3d, table, *, n_rows, hidden):
    per_w = n_rows // _NUM_WORKERS
    n_chunks = per_w // _CHUNK
    mesh = plsc.VectorSubcoreMesh(core_axis_name="c", subcore_axis_name="s")

    @functools.partial(
        pl.kernel,
        mesh=mesh,
        out_type=jax.ShapeDtypeStruct((n_rows, hidden), jnp.float32),
        scratch_types=(
            [pltpu.VMEM((n_chunks, _CHUNK), jnp.int32)]
            + [pltpu.VMEM((_CHUNK, hidden), jnp.float32)] * _NBUF
            + [pltpu.VMEM_SHARED((_NUM_SUBCORES, _NSLOT, _CHUNK, hidden),
                                 jnp.float32)]
            + [pltpu.SemaphoreType.DMA] * (2 * _NBUF + _NSLOT)
        ),
    )
    def emb_kernel(idx_hbm, table_hbm, out_hbm, idx_v, *bufs):
        rows = bufs[:_NBUF]
        slab = bufs[_NBUF]
        sems = bufs[_NBUF + 1:]
        gsem = sems[:_NBUF]
        wsem = sems[_NBUF:2 * _NBUF]
        dsem = sems[2 * _NBUF:]

        sid = lax.axis_index("s")
        wid = sid * _NUM_CORES + lax.axis_index("c")
        base = wid * per_w

        pltpu.sync_copy(idx_hbm.at[wid], idx_v)

        gcp = [None] * n_chunks
        wcp = [None] * n_chunks
        dcp = [None] * n_chunks
        dcp_waited = [False] * n_chunks

        def slot_of(g):
            return (g // 2) % _NSLOT

        def writeback(g):
            b = g % _NBUF
            gcp[g].wait()
            if g % 2 == 0:
                dst = out_hbm.at[pl.ds(base + g * _CHUNK, _CHUNK)]
                wcp[g] = pltpu.async_copy(rows[b], dst, wsem[b])
            else:
                s2 = slot_of(g)
                prev = g - 2 * _NSLOT
                if prev >= 0 and not dcp_waited[prev]:
                    dcp[prev].wait()
                    dcp_waited[prev] = True
                wcp[g] = pltpu.async_copy(rows[b], slab.at[sid, s2], wsem[b])

        def vacate(g):
            wcp[g].wait()
            if g % 2 == 1:
                s2 = slot_of(g)
                dcp[g] = pltpu.async_copy(
                    slab.at[sid, s2],
                    out_hbm.at[pl.ds(base + g * _CHUNK, _CHUNK)], dsem[s2])

        for g in range(n_chunks):
            b = g % _NBUF
            if g >= _NBUF:
                vacate(g - _NBUF)
            gcp[g] = pltpu.async_copy(table_hbm.at[idx_v.at[g]], rows[b], gsem[b])
            if g >= 1:
                writeback(g - 1)

        writeback(n_chunks - 1)
        for g in range(n_chunks - _NBUF, n_chunks):
            vacate(g)
        for g in range(n_chunks):
            if g % 2 == 1 and not dcp_waited[g]:
                dcp[g].wait()
                dcp_waited[g] = True

    return emb_kernel(positions3d, table)


def kernel(positions, pos_embedding):
    b, t = positions.shape
    n_rows = b * t
    hidden = pos_embedding.shape[1]
    pos3d = positions.reshape(
        _NUM_WORKERS, n_rows // _NUM_WORKERS // _CHUNK, _CHUNK
    ).astype(jnp.int32)
    out = _lookup(pos3d, pos_embedding, n_rows=n_rows, hidden=hidden)
    return out.reshape(b, t, hidden)
